# Initial kernel scaffold; baseline (speedup 1.0000x reference)
import functools

import jax
import jax.numpy as jnp
from jax import lax
from jax.experimental import pallas as pl
from jax.experimental.pallas import tpu as pltpu
from jax.experimental.pallas import tpu_sc as plsc

_N = 4096
_F = 128
_K = 20
_H1, _H2, _H3, _OUT = 128, 256, 512, 40
_R = 512
_NW = 32
_G = 4


def _edge_core(xr, xf, Wa_ref, Wb_ref, b_ref, idx_ref, p_ref, q_ref):
    gram = lax.dot_general(xr, xf, (((1,), (1,)), ((), ())),
                           preferred_element_type=jnp.float32)
    sqf = jnp.sum(xf * xf, axis=1)
    sqr = jnp.sum(xr * xr, axis=1)
    d2 = sqr[:, None] - 2.0 * gram + sqf[None, :]
    iota = lax.broadcasted_iota(jnp.int32, d2.shape, 1)
    lane_k = lax.broadcasted_iota(jnp.int32, (d2.shape[0], _K), 1)
    acc = jnp.zeros((d2.shape[0], _K), jnp.int32)
    m = jnp.full((d2.shape[0], 1), -jnp.inf, jnp.float32)
    for t in range(_K):
        masked = jnp.where(d2 > m, d2, jnp.inf)
        m = jnp.min(masked, axis=1, keepdims=True)
        it = jnp.min(jnp.where(d2 == m, iota, _N), axis=1)
        acc = jnp.where(lane_k == t, it[:, None], acc)
    idx_ref[...] = acc
    Wb = Wb_ref[...]
    p_ref[...] = jnp.dot(xr, Wa_ref[...] - Wb,
                         preferred_element_type=jnp.float32) + b_ref[...]
    q_ref[...] = jnp.dot(xr, Wb, preferred_element_type=jnp.float32)


def _edge1_body(xb_ref, xf_ref, Wa_ref, Wb_ref, b_ref, idx_ref, p_ref, q_ref):
    _edge_core(xb_ref[...], xf_ref[...], Wa_ref, Wb_ref, b_ref,
               idx_ref, p_ref, q_ref)


def _edge2_body(pb_ref, mb_ref, pf_ref, mf_ref, Wa_ref, Wb_ref, b_ref,
                idx_ref, p_ref, q_ref):
    xr = jnp.maximum(pb_ref[...] + mb_ref[...], 0.0)
    xf = jnp.maximum(pf_ref[...] + mf_ref[...], 0.0)
    _edge_core(xr, xf, Wa_ref, Wb_ref, b_ref, idx_ref, p_ref, q_ref)


def _full(shape):
    return pl.BlockSpec(shape, lambda i: tuple(0 for _ in shape))


def _rows(shape):
    return pl.BlockSpec(shape, lambda i: (i,) + tuple(0 for _ in shape[1:]))


def _edge1_call(x, Wa, Wb, b, H):
    grid = _N // _R
    return pl.pallas_call(
        _edge1_body,
        grid=(grid,),
        in_specs=[_rows((_R, _F)), _full((_N, _F)), _full((_F, H)),
                  _full((_F, H)), _full((1, H))],
        out_specs=[_rows((_R, _K)), _rows((_R, H)), _rows((_R, H))],
        out_shape=[jax.ShapeDtypeStruct((_N, _K), jnp.int32),
                   jax.ShapeDtypeStruct((_N, H), jnp.float32),
                   jax.ShapeDtypeStruct((_N, H), jnp.float32)],
    )(x, x, Wa, Wb, b)


def _edge2_call(P, M, Wa, Wb, b, H):
    grid = _N // _R
    return pl.pallas_call(
        _edge2_body,
        grid=(grid,),
        in_specs=[_rows((_R, _F)), _rows((_R, _F)), _full((_N, _F)),
                  _full((_N, _F)), _full((_F, H)), _full((_F, H)),
                  _full((1, H))],
        out_specs=[_rows((_R, _K)), _rows((_R, H)), _rows((_R, H))],
        out_shape=[jax.ShapeDtypeStruct((_N, _K), jnp.int32),
                   jax.ShapeDtypeStruct((_N, H), jnp.float32),
                   jax.ShapeDtypeStruct((_N, H), jnp.float32)],
    )(P, M, P, M, Wa, Wb, b)


def _make_gather_max(H):
    npw = _N // _NW
    ng = npw // _G
    il = _G * _K
    mesh = plsc.VectorSubcoreMesh(core_axis_name="c", subcore_axis_name="s")

    @functools.partial(
        pl.kernel, mesh=mesh,
        out_type=jax.ShapeDtypeStruct((_N, H), jnp.float32),
        scratch_types=[pltpu.VMEM((il,), jnp.int32),
                       pltpu.VMEM((il, H), jnp.float32),
                       pltpu.VMEM((_G, H), jnp.float32),
                       pltpu.SemaphoreType.DMA],
    )
    def gmax(idx_hbm, q_hbm, out_hbm, idx_v, rows_v, out_v, sem):
        wid = lax.axis_index("s") * 2 + lax.axis_index("c")

        def group(g, carry):
            base = wid * npw + g * _G
            pltpu.sync_copy(idx_hbm.at[pl.ds(base * _K, il)], idx_v)
            pltpu.async_copy(q_hbm.at[idx_v], rows_v, sem).wait()
            for n in range(_G):
                for c in range(H // 16):
                    sl = pl.ds(c * 16, 16)
                    a = rows_v[n * _K, sl]
                    for j in range(1, _K):
                        a = jnp.maximum(a, rows_v[n * _K + j, sl])
                    out_v[n, sl] = a
            pltpu.sync_copy(out_v, out_hbm.at[pl.ds(base, _G)])
            return carry

        lax.fori_loop(0, ng, group, 0)

    return gmax


_gmax1 = _make_gather_max(_H1)
_gmax2 = _make_gather_max(_H2)


def _head_body(pb_ref, mb_ref, W3_ref, b3_ref, W4_ref, b4_ref, out_ref):
    h2 = jnp.maximum(pb_ref[...] + mb_ref[...], 0.0)
    h3 = jnp.maximum(
        jnp.dot(h2, W3_ref[...], preferred_element_type=jnp.float32)
        + b3_ref[...], 0.0)
    z = jnp.dot(h3, W4_ref[...], preferred_element_type=jnp.float32) + b4_ref[...]
    out_ref[...] = 1.0 / (1.0 + jnp.exp(-z))


def _head_call(P2, M2, W3, b3, W4, b4):
    grid = _N // _R
    return pl.pallas_call(
        _head_body,
        grid=(grid,),
        in_specs=[_rows((_R, _H2)), _rows((_R, _H2)), _full((_H2, _H3)),
                  _full((1, _H3)), _full((_H3, _OUT)), _full((1, _OUT))],
        out_specs=_rows((_R, _OUT)),
        out_shape=jax.ShapeDtypeStruct((_N, _OUT), jnp.float32),
    )(P2, M2, W3, b3, W4, b4)


def kernel(x, W1, b1, W2, b2, W3, b3, W4, b4):
    idx1, P1, Q1 = _edge1_call(x, W1[:_F], W1[_F:], b1.reshape(1, -1), _H1)
    M1 = _gmax1(idx1.reshape(-1), Q1)
    idx2, P2, Q2 = _edge2_call(P1, M1, W2[:_H1], W2[_H1:],
                               b2.reshape(1, -1), _H2)
    M2 = _gmax2(idx2.reshape(-1), Q2)
    return _head_call(P2, M2, W3, b3.reshape(1, -1), W4, b4.reshape(1, -1))


# speedup vs baseline: 7.1200x; 7.1200x over previous
import functools

import jax
import jax.numpy as jnp
from jax import lax
from jax.experimental import pallas as pl
from jax.experimental.pallas import tpu as pltpu
from jax.experimental.pallas import tpu_sc as plsc

_N = 4096
_F = 128
_K = 20
_H1, _H2, _H3, _OUT = 128, 256, 512, 40
_R = 512
_NW = 32
_G = 4


def _edge_core(xr, xf, Wa_ref, Wb_ref, b_ref, idx_ref, p_ref, q_ref):
    gram = lax.dot_general(xr, xf, (((1,), (1,)), ((), ())),
                           preferred_element_type=jnp.float32)
    sqf = jnp.sum(xf * xf, axis=1)
    sqr = jnp.sum(xr * xr, axis=1)
    d2 = sqr[:, None] - 2.0 * gram + sqf[None, :]
    iota = lax.broadcasted_iota(jnp.int32, d2.shape, 1)
    lane_k = lax.broadcasted_iota(jnp.int32, (d2.shape[0], _K), 1)
    acc = jnp.zeros((d2.shape[0], _K), jnp.int32)
    m = jnp.full((d2.shape[0], 1), -jnp.inf, jnp.float32)
    for t in range(_K):
        masked = jnp.where(d2 > m, d2, jnp.inf)
        m = jnp.min(masked, axis=1, keepdims=True)
        it = jnp.min(jnp.where(d2 == m, iota, _N), axis=1)
        acc = jnp.where(lane_k == t, it[:, None], acc)
    idx_ref[...] = acc
    Wb = Wb_ref[...]
    p_ref[...] = jnp.dot(xr, Wa_ref[...] - Wb,
                         preferred_element_type=jnp.float32) + b_ref[...]
    q_ref[...] = jnp.dot(xr, Wb, preferred_element_type=jnp.float32)


def _edge1_body(xb_ref, xf_ref, Wa_ref, Wb_ref, b_ref, idx_ref, p_ref, q_ref):
    _edge_core(xb_ref[...], xf_ref[...], Wa_ref, Wb_ref, b_ref,
               idx_ref, p_ref, q_ref)


def _edge2_body(pb_ref, mb_ref, pf_ref, mf_ref, Wa_ref, Wb_ref, b_ref,
                idx_ref, p_ref, q_ref):
    xr = jnp.maximum(pb_ref[...] + mb_ref[...], 0.0)
    xf = jnp.maximum(pf_ref[...] + mf_ref[...], 0.0)
    _edge_core(xr, xf, Wa_ref, Wb_ref, b_ref, idx_ref, p_ref, q_ref)


def _full(shape):
    return pl.BlockSpec(shape, lambda i: tuple(0 for _ in shape))


def _rows(shape):
    return pl.BlockSpec(shape, lambda i: (i,) + tuple(0 for _ in shape[1:]))


def _edge1_call(x, Wa, Wb, b, H):
    grid = _N // _R
    return pl.pallas_call(
        _edge1_body,
        grid=(grid,),
        in_specs=[_rows((_R, _F)), _full((_N, _F)), _full((_F, H)),
                  _full((_F, H)), _full((1, H))],
        out_specs=[_rows((_R, _K)), _rows((_R, H)), _rows((_R, H))],
        out_shape=[jax.ShapeDtypeStruct((_N, _K), jnp.int32),
                   jax.ShapeDtypeStruct((_N, H), jnp.float32),
                   jax.ShapeDtypeStruct((_N, H), jnp.float32)],
    )(x, x, Wa, Wb, b)


def _edge2_call(P, M, Wa, Wb, b, H):
    grid = _N // _R
    return pl.pallas_call(
        _edge2_body,
        grid=(grid,),
        in_specs=[_rows((_R, _F)), _rows((_R, _F)), _full((_N, _F)),
                  _full((_N, _F)), _full((_F, H)), _full((_F, H)),
                  _full((1, H))],
        out_specs=[_rows((_R, _K)), _rows((_R, H)), _rows((_R, H))],
        out_shape=[jax.ShapeDtypeStruct((_N, _K), jnp.int32),
                   jax.ShapeDtypeStruct((_N, H), jnp.float32),
                   jax.ShapeDtypeStruct((_N, H), jnp.float32)],
    )(P, M, P, M, Wa, Wb, b)


@functools.lru_cache(maxsize=None)
def _make_gather_max(H):
    npw = _N // _NW
    ng = npw // _G
    il = _G * _K
    mesh = plsc.VectorSubcoreMesh(core_axis_name="c", subcore_axis_name="s")

    @functools.partial(
        pl.kernel, mesh=mesh,
        out_type=jax.ShapeDtypeStruct((_N, H), jnp.float32),
        scratch_types=[pltpu.VMEM((il,), jnp.int32),
                       pltpu.VMEM((il, H), jnp.float32),
                       pltpu.VMEM((_G, H), jnp.float32),
                       pltpu.SemaphoreType.DMA],
    )
    def gmax(idx_hbm, q_hbm, out_hbm, idx_v, rows_v, out_v, sem):
        wid = lax.axis_index("s") * 2 + lax.axis_index("c")

        def group(g, carry):
            base = wid * npw + g * _G
            pltpu.sync_copy(idx_hbm.at[pl.ds(base * _K, il)], idx_v)
            pltpu.async_copy(q_hbm.at[idx_v], rows_v, sem).wait()
            for n in range(_G):
                for c in range(H // 16):
                    sl = pl.ds(c * 16, 16)
                    a = rows_v[n * _K, sl]
                    for j in range(1, _K):
                        a = jnp.maximum(a, rows_v[n * _K + j, sl])
                    out_v[n, sl] = a
            pltpu.sync_copy(out_v, out_hbm.at[pl.ds(base, _G)])
            return carry

        lax.fori_loop(0, ng, group, 0)

    return gmax


def _head_body(pb_ref, mb_ref, W3_ref, b3_ref, W4_ref, b4_ref, out_ref):
    h2 = jnp.maximum(pb_ref[...] + mb_ref[...], 0.0)
    h3 = jnp.maximum(
        jnp.dot(h2, W3_ref[...], preferred_element_type=jnp.float32)
        + b3_ref[...], 0.0)
    z = jnp.dot(h3, W4_ref[...], preferred_element_type=jnp.float32) + b4_ref[...]
    out_ref[...] = 1.0 / (1.0 + jnp.exp(-z))


def _head_call(P2, M2, W3, b3, W4, b4):
    grid = _N // _R
    return pl.pallas_call(
        _head_body,
        grid=(grid,),
        in_specs=[_rows((_R, _H2)), _rows((_R, _H2)), _full((_H2, _H3)),
                  _full((1, _H3)), _full((_H3, _OUT)), _full((1, _OUT))],
        out_specs=_rows((_R, _OUT)),
        out_shape=jax.ShapeDtypeStruct((_N, _OUT), jnp.float32),
    )(P2, M2, W3, b3, W4, b4)


def kernel(x, W1, b1, W2, b2, W3, b3, W4, b4):
    idx1, P1, Q1 = _edge1_call(x, W1[:_F], W1[_F:], b1.reshape(1, -1), _H1)
    M1 = _make_gather_max(_H1)(idx1.reshape(-1), Q1)
    idx2, P2, Q2 = _edge2_call(P1, M1, W2[:_H1], W2[_H1:],
                               b2.reshape(1, -1), _H2)
    M2 = _make_gather_max(_H2)(idx2.reshape(-1), Q2)
    return _head_call(P2, M2, W3, b3.reshape(1, -1), W4, b4.reshape(1, -1))
